# Initial kernel scaffold; baseline (speedup 1.0000x reference)
import jax
import jax.numpy as jnp
from jax.experimental import pallas as pl


def kernel(node_list, nbr1, features, W_a1, b_a1, g_a1, be_a1, W_e1, b_e1, g_e1, be_e1, W_a2, b_a2, g_a2, be_a2, W_e2, b_e2, g_e2, be_e2):
    raise NotImplementedError("write your pallas kernel here")


# speedup vs baseline: 3.8809x; 3.8809x over previous
import functools

import jax
import jax.numpy as jnp
from jax import lax
from jax.experimental import pallas as pl
from jax.experimental.pallas import tpu as pltpu
from jax.experimental.pallas import tpu_sc as plsc

_EPS = 1e-5
_PAD = 16


def _fold_bn(W, b, g, be):
    s = g / jnp.sqrt(1.0 + _EPS)
    return W * s[None, :], b * s + be


def _pad_wb(W, b, out_w=_PAD):
    fin, fout = W.shape
    Wp = jnp.zeros((max(fin, out_w), out_w), W.dtype).at[:fin, :fout].set(W)
    bp = jnp.zeros((1, out_w), b.dtype).at[0, :fout].set(b)
    return Wp, bp


def _proj_body(x_ref, w_ref, b_ref, o_ref):
    x = x_ref[...]
    o_ref[...] = jnp.tanh(
        jnp.dot(x, w_ref[...], preferred_element_type=jnp.float32) + b_ref[...]
    )


def _project_all(features, Wp, bp, block_rows):
    N, D = features.shape
    grid = (N // block_rows,)
    return pl.pallas_call(
        _proj_body,
        grid=grid,
        in_specs=[
            pl.BlockSpec((block_rows, D), lambda i: (i, 0)),
            pl.BlockSpec((D, _PAD), lambda i: (0, 0)),
            pl.BlockSpec((1, _PAD), lambda i: (0, 0)),
        ],
        out_specs=pl.BlockSpec((block_rows, _PAD), lambda i: (i, 0)),
        out_shape=jax.ShapeDtypeStruct((N, _PAD), jnp.float32),
    )(features, Wp, bp)


def _gather_mean(P, idx_flat, n_seg, seg_len):
    info = plsc.get_sparse_core_info()
    NC, NS = info.num_cores, info.num_subcores
    NW = NC * NS
    seg_w = n_seg // NW
    rows_w = seg_w * seg_len
    inv = jnp.float32(1.0 / seg_len)

    mesh = plsc.VectorSubcoreMesh(core_axis_name="c", subcore_axis_name="s")

    @functools.partial(
        pl.kernel,
        out_type=jax.ShapeDtypeStruct((n_seg, _PAD), jnp.float32),
        mesh=mesh,
        compiler_params=pltpu.CompilerParams(use_tc_tiling_on_sc=False),
        scratch_types=[
            pltpu.VMEM((rows_w,), jnp.int32),
            pltpu.VMEM((rows_w, _PAD), jnp.float32),
            pltpu.VMEM((seg_w, _PAD), jnp.float32),
            pltpu.SemaphoreType.DMA,
        ],
    )
    def k(p_hbm, idx_hbm, out_hbm, idx_v, rows_v, out_v, sem):
        wid = lax.axis_index("s") * NC + lax.axis_index("c")
        pltpu.sync_copy(idx_hbm.at[pl.ds(wid * rows_w, rows_w)], idx_v)
        pltpu.async_copy(p_hbm.at[idx_v], rows_v, sem).wait()

        def body(s, _):
            base = s * seg_len
            acc = rows_v[base]
            for t in range(1, seg_len):
                acc = acc + rows_v[base + t]
            out_v[s] = acc * inv
            return 0

        lax.fori_loop(0, seg_w, body, 0)
        pltpu.sync_copy(out_v, out_hbm.at[pl.ds(wid * seg_w, seg_w)])

    return k(P, idx_flat)


def _tail_body(m1_ref, we1_ref, be1_ref, wa2_ref, ba2_ref, we2_ref, be2_ref,
               o_ref, *, n_j, batch):
    x = m1_ref[...]
    h1 = jnp.tanh(jnp.dot(x, we1_ref[...], preferred_element_type=jnp.float32)
                  + be1_ref[...])
    h2 = jnp.tanh(jnp.dot(h1, wa2_ref[...], preferred_element_type=jnp.float32)
                  + ba2_ref[...])
    m2 = h2[0:batch]
    for j in range(1, n_j):
        m2 = m2 + h2[j * batch:(j + 1) * batch]
    m2 = m2 * jnp.float32(1.0 / n_j)
    o_ref[...] = jnp.tanh(
        jnp.dot(m2, we2_ref[...], preferred_element_type=jnp.float32)
        + be2_ref[...]
    )


def _mlp_tail(M1, We1, be1, Wa2, ba2, We2, be2, n_j, batch):
    rows = M1.shape[0]
    full2 = pl.BlockSpec((_PAD, _PAD), lambda: (0, 0))
    fullb = pl.BlockSpec((1, _PAD), lambda: (0, 0))
    return pl.pallas_call(
        functools.partial(_tail_body, n_j=n_j, batch=batch),
        in_specs=[pl.BlockSpec((rows, _PAD), lambda: (0, 0)),
                  full2, fullb, full2, fullb, full2, fullb],
        out_specs=pl.BlockSpec((batch, _PAD), lambda: (0, 0)),
        out_shape=jax.ShapeDtypeStruct((batch, _PAD), jnp.float32),
    )(M1, We1, be1, Wa2, ba2, We2, be2)


def kernel(node_list, nbr1, features,
           W_a1, b_a1, g_a1, be_a1,
           W_e1, b_e1, g_e1, be_e1,
           W_a2, b_a2, g_a2, be_a2,
           W_e2, b_e2, g_e2, be_e2):
    del node_list
    B, S2p1, S1p1 = nbr1.shape
    N, D = features.shape

    Wa1f, ba1f = _fold_bn(W_a1, b_a1, g_a1, be_a1)
    Wa1p, ba1p = _pad_wb(Wa1f, ba1f)
    We1f, be1f = _fold_bn(W_e1, b_e1, g_e1, be_e1)
    We1p, be1p = _pad_wb(We1f, be1f)
    Wa2f, ba2f = _fold_bn(W_a2, b_a2, g_a2, be_a2)
    Wa2p, ba2p = _pad_wb(Wa2f, ba2f)
    We2f, be2f = _fold_bn(W_e2, b_e2, g_e2, be_e2)
    We2p, be2p = _pad_wb(We2f, be2f)

    P = _project_all(features, Wa1p, ba1p, block_rows=2000)

    idx_flat = jnp.transpose(nbr1, (1, 0, 2)).reshape(-1).astype(jnp.int32)

    M1 = _gather_mean(P, idx_flat, n_seg=S2p1 * B, seg_len=S1p1)

    out = _mlp_tail(M1, We1p, be1p, Wa2p, ba2p, We2p, be2p, n_j=S2p1, batch=B)
    return out[:, :W_e2.shape[1]] if W_e2.shape[1] != _PAD else out
